# baseline (speedup 1.0000x reference)
import functools

import jax
import jax.numpy as jnp
from jax import lax
from jax.experimental import pallas as pl
from jax.experimental.pallas import tpu as pltpu
from jax.experimental.pallas import tpu_sc as plsc

HID = 4096
K = 64
TOP_K = 8
ROWS_PER_TILE = 1024
SC_CHUNK = 128
NEG = -3.0e38


def _mm_body(h_ref, w_ref, bias_ref, out_ref):
    pt = lax.dot_general(
        w_ref[...], h_ref[...], (((1,), (1,)), ((), ())),
        preferred_element_type=jnp.float32,
        precision=jax.lax.Precision.DEFAULT)
    out_ref[...] = pt + bias_ref[...]


def _matmul_t(h2, w2, bias):
    rows = h2.shape[0]
    n_tiles = rows // ROWS_PER_TILE
    return pl.pallas_call(
        _mm_body,
        grid=(n_tiles,),
        in_specs=[
            pl.BlockSpec((ROWS_PER_TILE, HID), lambda i: (i, 0)),
            pl.BlockSpec((2 * K, HID), lambda i: (0, 0)),
            pl.BlockSpec((2 * K, 1), lambda i: (0, 0)),
        ],
        out_specs=pl.BlockSpec((2 * K, ROWS_PER_TILE), lambda i: (0, i)),
        out_shape=jax.ShapeDtypeStruct((2 * K, rows), jnp.float32),
    )(h2, w2, bias)


def _gate_kernel(rows):
    info = plsc.get_sparse_core_info()
    nw = info.num_cores * info.num_subcores
    rows_per_w = rows // nw
    n_chunks = rows_per_w // SC_CHUNK
    n_groups = SC_CHUNK // 16
    mesh = plsc.VectorSubcoreMesh(core_axis_name="c", subcore_axis_name="s")

    @functools.partial(
        pl.kernel, mesh=mesh,
        out_type=jax.ShapeDtypeStruct((rows,), jnp.float32),
        scratch_types=[
            pltpu.VMEM((2 * K, SC_CHUNK), jnp.float32),
            pltpu.VMEM((SC_CHUNK,), jnp.float32),
        ],
    )
    def gate(pt_hbm, out_hbm, buf, obuf):
        wid = lax.axis_index("s") * info.num_cores + lax.axis_index("c")
        wbase = wid * rows_per_w

        def chunk_body(c, _):
            base = wbase + c * SC_CHUNK
            pltpu.sync_copy(pt_hbm.at[:, pl.ds(base, SC_CHUNK)], buf)

            def group_body(g, _):
                col = g * 16

                def mx(k, m):
                    return jnp.maximum(m, buf[k, pl.ds(col, 16)])

                m1 = lax.fori_loop(0, K, mx, jnp.full((16,), NEG), unroll=16)
                t = m1
                for _p in range(TOP_K - 1):
                    def mx2(k, mp, tt=t):
                        v = buf[k, pl.ds(col, 16)]
                        return jnp.maximum(mp, jnp.where(v < tt, v, NEG))
                    t = lax.fori_loop(0, K, mx2, jnp.full((16,), NEG),
                                      unroll=16)

                def comb(k, carry):
                    acc, den = carry
                    v = buf[k, pl.ds(col, 16)]
                    e = buf[K + k, pl.ds(col, 16)]
                    w = jnp.where(v >= t, jnp.exp(v - m1), jnp.float32(0.0))
                    return acc + w * e, den + w

                zero = jnp.zeros((16,), jnp.float32)
                acc, den = lax.fori_loop(0, K, comb, (zero, zero), unroll=16)
                obuf[pl.ds(col, 16)] = acc / den
                return 0

            lax.fori_loop(0, n_groups, group_body, 0)
            pltpu.sync_copy(obuf, out_hbm.at[pl.ds(base, SC_CHUNK)])
            return 0

        lax.fori_loop(0, n_chunks, chunk_body, 0)

    return gate


N_PIPE = 4


def kernel(h, W_e, b_e, W_g):
    B, L, hid = h.shape
    rows = B * L
    h2 = h.reshape(rows, hid)
    w2 = jnp.concatenate([W_g, W_e], axis=0)
    bias = jnp.concatenate([jnp.zeros((K,), b_e.dtype), b_e]).reshape(2 * K, 1)
    cs = rows // N_PIPE
    gate = _gate_kernel(cs)
    outs = []
    for c in range(N_PIPE):
        pt = _matmul_t(h2[c * cs:(c + 1) * cs], w2, bias)
        outs.append(gate(pt))
    return jnp.concatenate(outs).reshape(B, L)


# speedup vs baseline: 2.4554x; 2.4554x over previous
import functools

import jax
import jax.numpy as jnp
from jax import lax
from jax.experimental import pallas as pl
from jax.experimental.pallas import tpu as pltpu
from jax.experimental.pallas import tpu_sc as plsc

HID = 4096
K = 64
TOP_K = 8
ROWS_PER_TILE = 1024
SC_CHUNK = 128
NEG = -3.0e38


def _mm_body(h_ref, w_ref, bias_ref, out_ref):
    pt = lax.dot_general(
        w_ref[...], h_ref[...], (((1,), (1,)), ((), ())),
        preferred_element_type=jnp.float32,
        precision=jax.lax.Precision.DEFAULT)
    out_ref[...] = pt + bias_ref[...]


def _matmul_t(h2, w2, bias):
    rows = h2.shape[0]
    n_tiles = rows // ROWS_PER_TILE
    return pl.pallas_call(
        _mm_body,
        grid=(n_tiles,),
        in_specs=[
            pl.BlockSpec((ROWS_PER_TILE, HID), lambda i: (i, 0)),
            pl.BlockSpec((2 * K, HID), lambda i: (0, 0)),
            pl.BlockSpec((2 * K, 1), lambda i: (0, 0)),
        ],
        out_specs=pl.BlockSpec((2 * K, ROWS_PER_TILE), lambda i: (0, i)),
        out_shape=jax.ShapeDtypeStruct((2 * K, rows), jnp.float32),
    )(h2, w2, bias)


def _gate_kernel(rows):
    info = plsc.get_sparse_core_info()
    nw = info.num_cores * info.num_subcores
    rows_per_w = rows // nw
    n_chunks = rows_per_w // SC_CHUNK
    n_groups = SC_CHUNK // 16
    mesh = plsc.VectorSubcoreMesh(core_axis_name="c", subcore_axis_name="s")

    @functools.partial(
        pl.kernel, mesh=mesh,
        out_type=jax.ShapeDtypeStruct((rows,), jnp.float32),
        scratch_types=[
            pltpu.VMEM((2 * K, SC_CHUNK), jnp.float32),
            pltpu.VMEM((SC_CHUNK,), jnp.float32),
        ],
    )
    def gate(pt_hbm, out_hbm, buf, obuf):
        wid = lax.axis_index("s") * info.num_cores + lax.axis_index("c")
        wbase = wid * rows_per_w

        def chunk_body(c, _):
            base = wbase + c * SC_CHUNK
            pltpu.sync_copy(pt_hbm.at[:, pl.ds(base, SC_CHUNK)], buf)

            def group_body(g, _):
                col = g * 16

                def ins(k, ts):
                    v = buf[k, pl.ds(col, 16)]
                    out = []
                    for j in range(TOP_K):
                        hi = jnp.maximum(ts[j], v)
                        v = jnp.minimum(ts[j], v)
                        out.append(hi)
                    return tuple(out)

                init = tuple(jnp.full((16,), NEG) for _ in range(TOP_K))
                ts = lax.fori_loop(0, K, ins, init, unroll=8)
                m1 = ts[0]
                t = ts[TOP_K - 1]

                def comb(k, carry):
                    acc, den = carry
                    v = buf[k, pl.ds(col, 16)]
                    e = buf[K + k, pl.ds(col, 16)]
                    w = jnp.where(v >= t, jnp.exp(v - m1), jnp.float32(0.0))
                    return acc + w * e, den + w

                zero = jnp.zeros((16,), jnp.float32)
                acc, den = lax.fori_loop(0, K, comb, (zero, zero), unroll=16)
                obuf[pl.ds(col, 16)] = acc / den
                return 0

            lax.fori_loop(0, n_groups, group_body, 0)
            pltpu.sync_copy(obuf, out_hbm.at[pl.ds(base, SC_CHUNK)])
            return 0

        lax.fori_loop(0, n_chunks, chunk_body, 0)

    return gate


def kernel(h, W_e, b_e, W_g):
    B, L, hid = h.shape
    rows = B * L
    h2 = h.reshape(rows, hid)
    w2 = jnp.concatenate([W_g, W_e], axis=0)
    bias = jnp.concatenate([jnp.zeros((K,), b_e.dtype), b_e]).reshape(2 * K, 1)
    pt = _matmul_t(h2, w2, bias)
    out = _gate_kernel(rows)(pt)
    return out.reshape(B, L)


# speedup vs baseline: 2.5392x; 1.0341x over previous
import functools

import jax
import jax.numpy as jnp
from jax import lax
from jax.experimental import pallas as pl
from jax.experimental.pallas import tpu as pltpu
from jax.experimental.pallas import tpu_sc as plsc

HID = 4096
K = 64
TOP_K = 8
ROWS_PER_TILE = 1024
SC_CHUNK = 128
NEG = -3.0e38


def _mm_body(h_ref, w_ref, bias_ref, out_ref):
    pt = lax.dot_general(
        w_ref[...], h_ref[...], (((1,), (1,)), ((), ())),
        preferred_element_type=jnp.float32,
        precision=jax.lax.Precision.DEFAULT)
    out_ref[...] = pt + bias_ref[...]


def _matmul_t(h2, w2, bias):
    rows = h2.shape[0]
    n_tiles = rows // ROWS_PER_TILE
    return pl.pallas_call(
        _mm_body,
        grid=(n_tiles,),
        in_specs=[
            pl.BlockSpec((ROWS_PER_TILE, HID), lambda i: (i, 0)),
            pl.BlockSpec((2 * K, HID), lambda i: (0, 0)),
            pl.BlockSpec((2 * K, 1), lambda i: (0, 0)),
        ],
        out_specs=pl.BlockSpec((2 * K, ROWS_PER_TILE), lambda i: (0, i)),
        out_shape=jax.ShapeDtypeStruct((2 * K, rows), jnp.float32),
    )(h2, w2, bias)


def _gate_kernel(rows):
    info = plsc.get_sparse_core_info()
    nw = info.num_cores * info.num_subcores
    rows_per_w = rows // nw
    n_chunks = rows_per_w // SC_CHUNK
    n_groups = SC_CHUNK // 16
    mesh = plsc.VectorSubcoreMesh(core_axis_name="c", subcore_axis_name="s")

    @functools.partial(
        pl.kernel, mesh=mesh,
        out_type=jax.ShapeDtypeStruct((rows,), jnp.float32),
        scratch_types=[
            pltpu.VMEM((2 * K, SC_CHUNK), jnp.float32),
            pltpu.VMEM((2 * K, SC_CHUNK), jnp.float32),
            pltpu.VMEM((rows // 32,), jnp.float32),
            pltpu.SemaphoreType.DMA,
            pltpu.SemaphoreType.DMA,
        ],
    )
    def gate(pt_hbm, out_hbm, buf0, buf1, obuf, sem0, sem1):
        wid = lax.axis_index("s") * info.num_cores + lax.axis_index("c")
        wbase = wid * rows_per_w
        bufs = (buf0, buf1)
        sems = (sem0, sem1)

        def start(c):
            return pltpu.async_copy(
                pt_hbm.at[:, pl.ds(wbase + c * SC_CHUNK, SC_CHUNK)],
                bufs[c % 2], sems[c % 2])

        pending = start(0)
        for c in range(n_chunks):
            buf = bufs[c % 2]
            nxt = start(c + 1) if c + 1 < n_chunks else None
            pending.wait()
            pending = nxt

            def group_body(g, _, buf=buf, c=c):
                col = g * 16

                def ins(k, ts):
                    v = buf[k, pl.ds(col, 16)]
                    out = []
                    for j in range(TOP_K):
                        hi = jnp.maximum(ts[j], v)
                        v = jnp.minimum(ts[j], v)
                        out.append(hi)
                    return tuple(out)

                init = tuple(jnp.full((16,), NEG) for _ in range(TOP_K))
                ts = lax.fori_loop(0, K, ins, init, unroll=8)
                m1 = ts[0]
                t = ts[TOP_K - 1]

                def comb(k, carry):
                    acc, den = carry
                    v = buf[k, pl.ds(col, 16)]
                    e = buf[K + k, pl.ds(col, 16)]
                    w = jnp.where(v >= t, jnp.exp(v - m1), jnp.float32(0.0))
                    return acc + w * e, den + w

                zero = jnp.zeros((16,), jnp.float32)
                acc, den = lax.fori_loop(0, K, comb, (zero, zero), unroll=16)
                obuf[pl.ds(c * SC_CHUNK + col, 16)] = acc / den
                return 0

            lax.fori_loop(0, n_groups, group_body, 0)

        pltpu.sync_copy(obuf, out_hbm.at[pl.ds(wbase, rows_per_w)])

    return gate


def kernel(h, W_e, b_e, W_g):
    B, L, hid = h.shape
    rows = B * L
    h2 = h.reshape(rows, hid)
    w2 = jnp.concatenate([W_g, W_e], axis=0)
    bias = jnp.concatenate([jnp.zeros((K,), b_e.dtype), b_e]).reshape(2 * K, 1)
    pt = _matmul_t(h2, w2, bias)
    out = _gate_kernel(rows)(pt)
    return out.reshape(B, L)
